# baseline (speedup 1.0000x reference)
import functools

import jax
import jax.numpy as jnp
from jax import lax
from jax.experimental import pallas as pl
from jax.experimental.pallas import tpu as pltpu
from jax.experimental.pallas import tpu_sc as plsc

_B = 16384
_D = 64
_NC = 2
_NS = 16
_NW = _NC * _NS
_BPW = _B // _NW
_CHUNK = 128
_NCHUNK = _BPW // _CHUNK
_GROUPS = _BPW // 16


def _distmult_body(head_hbm, rel_hbm, tail_hbm, entity_hbm, relation_hbm,
                   out_hbm, hidx, ridx, tidx, h_rows, r_rows, t_rows,
                   out_buf, sem):
    wid = lax.axis_index("s") * _NC + lax.axis_index("c")
    idx_base = wid * _NCHUNK

    pltpu.sync_copy(head_hbm.at[pl.ds(idx_base, _NCHUNK)], hidx)
    pltpu.sync_copy(rel_hbm.at[pl.ds(idx_base, _NCHUNK)], ridx)
    pltpu.sync_copy(tail_hbm.at[pl.ds(idx_base, _NCHUNK)], tidx)

    copies = []
    for j in range(_NCHUNK):
        dst = pl.ds(j * _CHUNK, _CHUNK)
        copies.append(pltpu.async_copy(entity_hbm.at[hidx.at[j]],
                                       h_rows.at[dst], sem))
        copies.append(pltpu.async_copy(relation_hbm.at[ridx.at[j]],
                                       r_rows.at[dst], sem))
        copies.append(pltpu.async_copy(entity_hbm.at[tidx.at[j]],
                                       t_rows.at[dst], sem))
    for c in copies:
        c.wait()

    iota16 = lax.iota(jnp.int32, 16)

    def group(g, carry):
        row0 = g * 16
        tot = jnp.zeros((16,), jnp.float32)
        for jj in range(16):
            row = row0 + jj
            acc = None
            for c in range(_D // 16):
                sl = pl.ds(c * 16, 16)
                p = h_rows[row, sl] * r_rows[row, sl] * t_rows[row, sl]
                acc = p if acc is None else acc + p
            tot = jnp.where(iota16 == jj, jnp.sum(acc), tot)
        out_buf[pl.ds(row0, 16)] = tot
        return carry

    lax.fori_loop(0, _GROUPS, group, 0)

    pltpu.sync_copy(out_buf, out_hbm.at[pl.ds(wid * _BPW, _BPW)])


_distmult_sc = functools.partial(
    pl.kernel,
    out_type=jax.ShapeDtypeStruct((_B,), jnp.float32),
    scratch_types=[
        pltpu.VMEM((_NCHUNK, _CHUNK), jnp.int32),
        pltpu.VMEM((_NCHUNK, _CHUNK), jnp.int32),
        pltpu.VMEM((_NCHUNK, _CHUNK), jnp.int32),
        pltpu.VMEM((_BPW, _D), jnp.float32),
        pltpu.VMEM((_BPW, _D), jnp.float32),
        pltpu.VMEM((_BPW, _D), jnp.float32),
        pltpu.VMEM((_BPW,), jnp.float32),
        pltpu.SemaphoreType.DMA,
    ],
    mesh=plsc.VectorSubcoreMesh(core_axis_name="c", subcore_axis_name="s"),
    compiler_params=pltpu.CompilerParams(
        needs_layout_passes=False, use_tc_tiling_on_sc=False),
)(_distmult_body)


@jax.jit
def kernel(head_idx, rel_idx, tail_idx, entity_table, relation_table):
    h2 = head_idx.reshape(_NW * _NCHUNK, _CHUNK)
    r2 = rel_idx.reshape(_NW * _NCHUNK, _CHUNK)
    t2 = tail_idx.reshape(_NW * _NCHUNK, _CHUNK)
    return _distmult_sc(h2, r2, t2, entity_table, relation_table)


# speedup vs baseline: 2.2463x; 2.2463x over previous
import functools

import jax
import jax.numpy as jnp
from jax import lax
from jax.experimental import pallas as pl
from jax.experimental.pallas import tpu as pltpu
from jax.experimental.pallas import tpu_sc as plsc

_B = 16384
_D = 64
_TR = 8
_NTILES = 1000000 // _TR
_NC = 2
_NS = 16
_NW = _NC * _NS
_BPW = _B // _NW
_PASS = 128
_NPASS = _BPW // _PASS


def _distmult_body(htile_hbm, ttile_hbm, rtile_hbm, hsub_hbm, tsub_hbm,
                   rsub_hbm, entity_hbm, rel_hbm, out_hbm,
                   htile, ttile, rtile, hsub, tsub, rsub,
                   h_rows, t_rows, r_rows, out_buf, sem):
    wid = lax.axis_index("s") * _NC + lax.axis_index("c")

    pltpu.sync_copy(htile_hbm.at[wid], htile)
    pltpu.sync_copy(ttile_hbm.at[wid], ttile)
    pltpu.sync_copy(rtile_hbm.at[wid], rtile)
    pltpu.sync_copy(hsub_hbm.at[wid], hsub)
    pltpu.sync_copy(tsub_hbm.at[wid], tsub)
    pltpu.sync_copy(rsub_hbm.at[wid], rsub)

    iota16 = lax.iota(jnp.int32, 16)

    def do_pass(p, carry):
        row0 = p * _PASS
        for half in range(_PASS // 64):
            for q in range(4):
                base = row0 + half * 64 + q * 16
                sl = pl.ds(base, 16)
                hv, hs = htile[sl], hsub[sl]
                tv, ts = ttile[sl], tsub[sl]
                rv, rs = rtile[sl], rsub[sl]
                for i in range(16):
                    lj = half * 64 + q * 16 + i
                    a, b = lj >> 3, lj & 7
                    pltpu.async_copy(entity_hbm.at[hv[i], hs[i]],
                                     h_rows.at[a, b], sem)
                    pltpu.async_copy(entity_hbm.at[tv[i], ts[i]],
                                     t_rows.at[a, b], sem)
                    pltpu.async_copy(rel_hbm.at[rv[i], rs[i]],
                                     r_rows.at[a, b], sem)
            blk = pl.ds(half * 8, 8)
            src = entity_hbm.at[pl.ds(0, 8)]
            rsrc = rel_hbm.at[pl.ds(0, 8)]
            pltpu.make_async_copy(src, h_rows.at[blk], sem).wait()
            pltpu.make_async_copy(src, t_rows.at[blk], sem).wait()
            pltpu.make_async_copy(rsrc, r_rows.at[blk], sem).wait()

        for g in range(_PASS // 16):
            tot = jnp.zeros((16,), jnp.float32)
            for jj in range(16):
                lj = g * 16 + jj
                a, b = lj >> 3, lj & 7
                acc = None
                for c in range(_D // 16):
                    sl = pl.ds(c * 16, 16)
                    prod = (h_rows[a, b, sl] * r_rows[a, b, sl]
                            * t_rows[a, b, sl])
                    acc = prod if acc is None else acc + prod
                tot = jnp.where(iota16 == jj, jnp.sum(acc), tot)
            out_buf[pl.ds(row0 + g * 16, 16)] = tot
        return carry

    lax.fori_loop(0, _NPASS, do_pass, 0)

    pltpu.sync_copy(out_buf, out_hbm.at[pl.ds(wid * _BPW, _BPW)])


_distmult_sc = functools.partial(
    pl.kernel,
    out_type=jax.ShapeDtypeStruct((_B,), jnp.float32),
    scratch_types=[
        pltpu.VMEM((_BPW,), jnp.int32),
        pltpu.VMEM((_BPW,), jnp.int32),
        pltpu.VMEM((_BPW,), jnp.int32),
        pltpu.VMEM((_BPW,), jnp.int32),
        pltpu.VMEM((_BPW,), jnp.int32),
        pltpu.VMEM((_BPW,), jnp.int32),
        pltpu.VMEM((_PASS // _TR, _TR, _D), jnp.float32),
        pltpu.VMEM((_PASS // _TR, _TR, _D), jnp.float32),
        pltpu.VMEM((_PASS // _TR, _TR, _D), jnp.float32),
        pltpu.VMEM((_BPW,), jnp.float32),
        pltpu.SemaphoreType.DMA,
    ],
    mesh=plsc.VectorSubcoreMesh(core_axis_name="c", subcore_axis_name="s"),
    compiler_params=pltpu.CompilerParams(needs_layout_passes=False),
)(_distmult_body)


@jax.jit
def kernel(head_idx, rel_idx, tail_idx, entity_table, relation_table):
    h2, r2, t2 = (x.reshape(_NW, _BPW) for x in (head_idx, rel_idx, tail_idx))
    et3 = entity_table.reshape(_NTILES, _TR, _D)
    rt3 = relation_table.reshape(1000 // _TR, _TR, _D)
    return _distmult_sc(h2 >> 3, t2 >> 3, r2 >> 3, h2 & 7, t2 & 7, r2 & 7,
                        et3, rt3)
